# Initial kernel scaffold; baseline (speedup 1.0000x reference)
import jax
import jax.numpy as jnp
from jax.experimental import pallas as pl


def kernel(x, emb_mask, mask_indices):
    raise NotImplementedError("write your pallas kernel here")


# speedup vs baseline: 2.4715x; 2.4715x over previous
import functools

import jax
import jax.numpy as jnp
from jax import lax
from jax.experimental import pallas as pl
from jax.experimental.pallas import tpu as pltpu
from jax.experimental.pallas import tpu_sc as plsc

_B, _N, _DIM = 4, 8192, 1024
_M = 1228
_LANES = 16
_M_CHUNKS = -(-_M // _LANES)
_M_PAD = _M_CHUNKS * _LANES
_NC, _NS = 2, 16
_PARTS = (_NC * _NS) // _B
_ROWS = _N // _PARTS

_R = 1024
_NB = _N // _R


def _sc_rowmask_body(idx_hbm, mask_hbm, idx_v, mask_v):
    c = lax.axis_index("c")
    s = lax.axis_index("s")
    wid = s * _NC + c
    b = wid // _PARTS
    part = wid - b * _PARTS
    lo = part * _ROWS

    pltpu.sync_copy(idx_hbm.at[pl.ds(b * _M_PAD, _M_PAD)], idx_v)

    zeros = jnp.zeros((_LANES,), jnp.float32)

    def _zero(i, carry):
        mask_v[pl.ds(i * _LANES, _LANES)] = zeros
        return carry

    lax.fori_loop(0, _ROWS // _LANES, _zero, 0)

    ones = jnp.ones((_LANES,), jnp.float32)
    lanes = lax.iota(jnp.int32, _LANES)

    def _scat(i, carry):
        idx = idx_v[pl.ds(i * _LANES, _LANES)]
        valid = (i * _LANES + lanes) < _M
        inr = valid & (idx >= lo) & (idx < lo + _ROWS)
        plsc.store_scatter(mask_v, [idx - lo], ones, mask=inr)
        return carry

    lax.fori_loop(0, _M_CHUNKS, _scat, 0)

    pltpu.sync_copy(mask_v, mask_hbm.at[wid, 0])


@functools.lru_cache(maxsize=None)
def _sc_rowmask_fn():
    return pl.kernel(
        _sc_rowmask_body,
        out_type=jax.ShapeDtypeStruct((_B * _NB, 1, _ROWS), jnp.float32),
        mesh=plsc.VectorSubcoreMesh(core_axis_name="c", subcore_axis_name="s",
                                    num_cores=_NC, num_subcores=_NS),
        scratch_types=[
            pltpu.VMEM((_M_PAD,), jnp.int32),
            pltpu.VMEM((_ROWS,), jnp.float32),
        ],
        compiler_params=pltpu.CompilerParams(needs_layout_passes=False),
    )


def _tc_select_body(mask_ref, emb_ref, x_ref, o_ref):
    m = mask_ref[...]
    sel = m.reshape(_R, 1) > 0.0
    o_ref[0] = jnp.where(sel, emb_ref[...], x_ref[0])


def _tc_select(rowmask, emb_mask, x):
    return pl.pallas_call(
        _tc_select_body,
        grid=(_B, _NB),
        in_specs=[
            pl.BlockSpec((1, 1, _R), lambda b, j: (b * _NB + j, 0, 0)),
            pl.BlockSpec((1, _DIM), lambda b, j: (0, 0)),
            pl.BlockSpec((1, _R, _DIM), lambda b, j: (b, j, 0)),
        ],
        out_specs=pl.BlockSpec((1, _R, _DIM), lambda b, j: (b, j, 0)),
        out_shape=jax.ShapeDtypeStruct((_B, _N, _DIM), jnp.float32),
    )(rowmask, emb_mask, x)


def kernel(x, emb_mask, mask_indices):
    idx_flat = jnp.pad(mask_indices, ((0, 0), (0, _M_PAD - _M))).reshape(-1)
    rowmask = _sc_rowmask_fn()(idx_flat)
    masked = _tc_select(rowmask, emb_mask, x)
    return masked, mask_indices


# speedup vs baseline: 2.5099x; 1.0155x over previous
import functools

import jax
import jax.numpy as jnp
from jax import lax
from jax.experimental import pallas as pl
from jax.experimental.pallas import tpu as pltpu
from jax.experimental.pallas import tpu_sc as plsc

_B, _N, _DIM = 4, 8192, 1024
_M = 1228
_LANES = 16
_M_CHUNKS = -(-_M // _LANES)
_M_PAD = _M_CHUNKS * _LANES
_NC, _NS = 2, 16
_PARTS = (_NC * _NS) // _B
_ROWS = _N // _PARTS

_R = 2048
_NB = _N // _R
_CHUNKS_PER_BLOCK = _R // _ROWS


def _sc_rowmask_body(idx_hbm, mask_hbm, idx_v, mask_v):
    c = lax.axis_index("c")
    s = lax.axis_index("s")
    wid = s * _NC + c
    b = wid // _PARTS
    part = wid - b * _PARTS
    lo = part * _ROWS

    pltpu.sync_copy(idx_hbm.at[pl.ds(b * _M_PAD, _M_PAD)], idx_v)

    zeros = jnp.zeros((_LANES,), jnp.float32)

    def _zero(i, carry):
        mask_v[pl.ds(i * _LANES, _LANES)] = zeros
        return carry

    lax.fori_loop(0, _ROWS // _LANES, _zero, 0)

    ones = jnp.ones((_LANES,), jnp.float32)
    lanes = lax.iota(jnp.int32, _LANES)

    def _scat(i, carry):
        idx = idx_v[pl.ds(i * _LANES, _LANES)]
        valid = (i * _LANES + lanes) < _M
        inr = valid & (idx >= lo) & (idx < lo + _ROWS)
        plsc.store_scatter(mask_v, [idx - lo], ones, mask=inr)
        return carry

    lax.fori_loop(0, _M_CHUNKS, _scat, 0)

    pltpu.sync_copy(mask_v, mask_hbm.at[wid, 0])


@functools.lru_cache(maxsize=None)
def _sc_rowmask_fn():
    return pl.kernel(
        _sc_rowmask_body,
        out_type=jax.ShapeDtypeStruct((_B * _NB, 1, _ROWS), jnp.float32),
        mesh=plsc.VectorSubcoreMesh(core_axis_name="c", subcore_axis_name="s",
                                    num_cores=_NC, num_subcores=_NS),
        scratch_types=[
            pltpu.VMEM((_M_PAD,), jnp.int32),
            pltpu.VMEM((_ROWS,), jnp.float32),
        ],
        compiler_params=pltpu.CompilerParams(needs_layout_passes=False),
    )


def _tc_select_body(mask_ref, emb_ref, x_ref, o_ref):
    emb = emb_ref[...]
    for k in range(_CHUNKS_PER_BLOCK):
        m = mask_ref[k]
        sel = m.reshape(_ROWS, 1) > 0.0
        rows = pl.ds(k * _ROWS, _ROWS)
        o_ref[0, rows, :] = jnp.where(sel, emb, x_ref[0, rows, :])


def _tc_select(rowmask, emb_mask, x):
    return pl.pallas_call(
        _tc_select_body,
        grid=(_B, _NB),
        in_specs=[
            pl.BlockSpec((_CHUNKS_PER_BLOCK, 1, _ROWS),
                         lambda b, j: (b * _NB + j, 0, 0)),
            pl.BlockSpec((1, _DIM), lambda b, j: (0, 0)),
            pl.BlockSpec((1, _R, _DIM), lambda b, j: (b, j, 0)),
        ],
        out_specs=pl.BlockSpec((1, _R, _DIM), lambda b, j: (b, j, 0)),
        out_shape=jax.ShapeDtypeStruct((_B, _N, _DIM), jnp.float32),
    )(rowmask, emb_mask, x)


def kernel(x, emb_mask, mask_indices):
    idx_flat = jnp.pad(mask_indices, ((0, 0), (0, _M_PAD - _M))).reshape(-1)
    rowmask = _sc_rowmask_fn()(idx_flat)
    masked = _tc_select(rowmask, emb_mask, x)
    return masked, mask_indices
